# Initial kernel scaffold; baseline (speedup 1.0000x reference)
import jax
import jax.numpy as jnp
from jax.experimental import pallas as pl


def kernel(y_true, y_pred):
    raise NotImplementedError("write your pallas kernel here")


# speedup vs baseline: 505.1010x; 505.1010x over previous
import functools

import jax
import jax.numpy as jnp
from jax import lax
from jax.experimental import pallas as pl
from jax.experimental.pallas import tpu as pltpu

_NEG_POS = 3.0
_H = 512
_W = 512
_N = float(_H * _W)


def _sl1_of_mag(x):
    return jnp.where(x < 1.0, 0.5 * x * x, x - 0.5)


def _field_block(fields):
    ii = lax.broadcasted_iota(jnp.int32, (8, 128), 0)
    out = jnp.zeros((8, 128), jnp.float32)
    for r, f in enumerate(fields):
        out = out + jnp.where(ii == r, f, 0.0)
    return out


def _main_body(yt_ref, yp_ref, acc_ref):
    b = pl.program_id(0)
    c = pl.program_id(1)

    yt = yt_ref[0, 0]
    yp = yp_ref[0, 0]
    d = yt - yp
    ad = jnp.abs(d)
    sl1 = _sl1_of_mag(ad)
    posf = (yt >= 0.5).astype(jnp.float32)
    negf = 1.0 - posf
    loss = ad * negf
    z = (loss == 0.0).astype(jnp.float32)
    psl1 = sl1 * posf
    floss = sl1 * negf

    rz = jnp.sum(z, axis=1, keepdims=True)
    rp = jnp.sum(psl1, axis=1, keepdims=True)
    rS = jnp.sum(floss, axis=1, keepdims=True)
    rpos = jnp.sum(posf, axis=1, keepdims=True)

    num_pos = jnp.sum(rpos)
    pos_sl1 = jnp.sum(rp)
    S_nz = jnp.sum(rS)
    nz = _N - jnp.sum(rz)
    k = jnp.minimum(_NEG_POS * num_pos, _N - 1.0)
    m = k - nz

    ii = lax.broadcasted_iota(jnp.int32, (_H, _H), 0)
    jj = lax.broadcasted_iota(jnp.int32, (_H, _H), 1)
    tlow = (jj < ii).astype(jnp.float32)
    ro = jnp.floor(
        jnp.dot(tlow, rz, preferred_element_type=jnp.float32) + 0.5)

    full = ((ro + rz) <= m).astype(jnp.float32)
    rstar = jnp.sum(full)
    fullsum = jnp.sum(rp * full)

    ri = lax.broadcasted_iota(jnp.int32, (_H, 1), 0)
    rowmask = (ri == rstar.astype(jnp.int32)).astype(jnp.float32)
    zrow = jnp.sum(z * rowmask, axis=0, keepdims=True)
    prow = jnp.sum(psl1 * rowmask, axis=0, keepdims=True)
    ro_r = jnp.sum(ro * rowmask)

    ew = jnp.floor(
        jnp.dot(zrow, (ii < jj).astype(jnp.float32),
                preferred_element_type=jnp.float32) + 0.5)
    partial = jnp.sum(prow * ((ro_r + ew) < m).astype(jnp.float32))

    negB = jnp.where(k > 0.0, S_nz + fullsum + partial, 0.0)
    needA = jnp.logical_and(k <= nz, k > 0.0).astype(jnp.float32)

    contrib = _field_block([num_pos, k, pos_sl1, negB, needA])

    @pl.when(jnp.logical_and(b == 0, c == 0))
    def _():
        acc_ref[...] = jnp.zeros_like(acc_ref)

    acc_ref[...] += contrib


def _fallback_body(yt_ref, yp_ref, acc_ref):
    b = pl.program_id(0)
    c = pl.program_id(1)

    yt = yt_ref[0, 0]
    yp = yp_ref[0, 0]
    ad = jnp.abs(yt - yp)
    sl1 = _sl1_of_mag(ad)
    posf = (yt >= 0.5).astype(jnp.float32)
    negf = 1.0 - posf
    loss = ad * negf
    floss = sl1 * negf

    num_pos = jnp.sum(posf)
    nz = jnp.sum((loss > 0.0).astype(jnp.float32))
    S_nz = jnp.sum(floss)
    k = jnp.minimum(_NEG_POS * num_pos, _N - 1.0)
    needA = jnp.logical_and(k <= nz, k > 0.0)

    bits = lax.bitcast_convert_type(loss, jnp.int32)

    def body(i, lo):
        cand = lo | (1 << (30 - i)).astype(jnp.int32)
        cnt = jnp.sum((bits >= cand).astype(jnp.float32))
        return jnp.where(cnt >= k, cand, lo)

    tbits = lax.fori_loop(0, 31, body, jnp.int32(0))
    t = lax.bitcast_convert_type(tbits, jnp.float32)
    gt = (bits > tbits).astype(jnp.float32)
    cnt_gt = jnp.sum(gt)
    sum_gt = jnp.sum(floss * gt)
    negA = sum_gt + (k - cnt_gt) * _sl1_of_mag(t)
    delta = jnp.where(needA, negA - S_nz, 0.0)

    contrib = _field_block([delta])

    @pl.when(jnp.logical_and(b == 0, c == 0))
    def _():
        acc_ref[...] = jnp.zeros_like(acc_ref)

    acc_ref[...] += contrib


def _plane_call(body, y_true, y_pred):
    B, C, H, W = y_true.shape
    return pl.pallas_call(
        body,
        grid=(B, C),
        in_specs=[
            pl.BlockSpec((1, 1, H, W), lambda b, c: (b, c, 0, 0)),
            pl.BlockSpec((1, 1, H, W), lambda b, c: (b, c, 0, 0)),
        ],
        out_specs=pl.BlockSpec((8, 128), lambda b, c: (0, 0)),
        out_shape=jax.ShapeDtypeStruct((8, 128), jnp.float32),
        compiler_params=pltpu.CompilerParams(
            dimension_semantics=("arbitrary", "arbitrary")),
    )(y_true, y_pred)


@jax.jit
def kernel(y_true, y_pred):
    acc = _plane_call(_main_body, y_true, y_pred)
    pos_cnt = jnp.maximum(acc[0, 0], 1.0)
    neg_cnt = jnp.maximum(acc[1, 0], 1.0)
    delta = lax.cond(
        acc[4, 0] > 0.5,
        lambda: _plane_call(_fallback_body, y_true, y_pred)[0, 0],
        lambda: jnp.float32(0.0),
    )
    return _NEG_POS * acc[2, 0] / pos_cnt + (acc[3, 0] + delta) / neg_cnt
